# baseline (speedup 1.0000x reference)
import functools

import jax
import jax.numpy as jnp
from jax import lax
from jax.experimental import pallas as pl
from jax.experimental.pallas import tpu as pltpu
from jax.experimental.pallas import tpu_sc as plsc

N = 32768
B = 128
NC = 2
NS = 16
NW = NC * NS
LANES = 16
ROWS_PER_W = N // NW
CHUNK = 512
NCHUNK = ROWS_PER_W // CHUNK
VPR = B // LANES
COLS_PER_W = B // NW
GROUPS_PER_VREG = LANES // COLS_PER_W
ZCHUNK = 4096
BIG = 9999999999.9

_mesh = plsc.VectorSubcoreMesh(core_axis_name="c", subcore_axis_name="s")


def _wid():
    return lax.axis_index("s") * NC + lax.axis_index("c")


@functools.partial(
    pl.kernel,
    out_type=(
        jax.ShapeDtypeStruct((NW, B), jnp.float32),
        jax.ShapeDtypeStruct((NW, B), jnp.int32),
    ),
    mesh=_mesh,
    scratch_types=[
        pltpu.VMEM((CHUNK, B), jnp.float32),
        pltpu.VMEM((B,), jnp.float32),
        pltpu.VMEM((B,), jnp.int32),
    ],
)
def _phase1(x_hbm, mins_hbm, idxs_hbm, buf, minb, idxb):
    w = _wid()
    row0 = w * ROWS_PER_W

    mins = [jnp.full((LANES,), jnp.float32(jnp.inf)) for _ in range(VPR)]
    idxs = [jnp.zeros((LANES,), jnp.int32) for _ in range(VPR)]
    carry = tuple(mins) + tuple(idxs)

    for g in range(NCHUNK):
        pltpu.sync_copy(
            x_hbm.at[pl.ds(N + row0 + g * CHUNK, CHUNK), :], buf)

        def row_body(r, c, g=g):
            ms = list(c[:VPR])
            ix = list(c[VPR:])
            ridx = row0 + g * CHUNK + r
            for j in range(VPR):
                v = buf[r, pl.ds(j * LANES, LANES)]
                vz = jnp.where(v == jnp.float32(0.0), BIG, v)
                pred = vz < ms[j]
                ms[j] = jnp.where(pred, vz, ms[j])
                ix[j] = jnp.where(pred, ridx, ix[j])
            return tuple(ms) + tuple(ix)

        carry = lax.fori_loop(0, CHUNK, row_body, carry)

    for j in range(VPR):
        minb[pl.ds(j * LANES, LANES)] = carry[j]
        idxb[pl.ds(j * LANES, LANES)] = carry[VPR + j]
    pltpu.sync_copy(minb, mins_hbm.at[w])
    pltpu.sync_copy(idxb, idxs_hbm.at[w])


@functools.partial(
    pl.kernel,
    out_type=jax.ShapeDtypeStruct((B, N), jnp.float32),
    mesh=_mesh,
    scratch_types=[
        pltpu.VMEM((NW, B), jnp.float32),
        pltpu.VMEM((NW, B), jnp.int32),
        pltpu.VMEM((ZCHUNK,), jnp.float32),
        pltpu.VMEM((LANES,), jnp.float32),
        pltpu.VMEM((2 * LANES,), jnp.int32),
    ],
)
def _phase2(mins_hbm, idxs_hbm, out_hbm, vals, idxv, zbuf, ovec, mbuf):
    w = _wid()

    pltpu.sync_copy(mins_hbm, vals)
    pltpu.sync_copy(idxs_hbm, idxv)

    zv = jnp.zeros((LANES,), jnp.float32)

    def zero_body(i, c):
        zbuf[pl.ds(i * LANES, LANES)] = zv
        return c

    lax.fori_loop(0, ZCHUNK // LANES, zero_body, 0)

    c16 = (w // GROUPS_PER_VREG) * LANES
    minv = jnp.full((LANES,), jnp.float32(jnp.inf))
    mini = jnp.zeros((LANES,), jnp.int32)
    for w2 in range(NW):
        v = vals[w2, pl.ds(c16, LANES)]
        iv = idxv[w2, pl.ds(c16, LANES)]
        pred = v < minv
        minv = jnp.where(pred, v, minv)
        mini = jnp.where(pred, iv, mini)

    mbuf[pl.ds(0, LANES)] = mini
    mbuf[pl.ds(LANES, LANES)] = jnp.zeros((LANES,), jnp.int32)
    lane_iota = lax.iota(jnp.int32, LANES)
    for k in range(COLS_PER_W):
        row = w * COLS_PER_W + k
        lane = (w % GROUPS_PER_VREG) * COLS_PER_W + k
        idx_scalar = mbuf[pl.ds(lane, LANES)][0]
        for ch in range(N // ZCHUNK):
            pltpu.sync_copy(zbuf, out_hbm.at[row, pl.ds(ch * ZCHUNK, ZCHUNK)])
        base = pl.multiple_of((idx_scalar // LANES) * LANES, LANES)
        ovec[...] = jnp.where(
            lane_iota == idx_scalar - base, jnp.float32(1.0), jnp.float32(0.0))
        pltpu.sync_copy(ovec, out_hbm.at[row, pl.ds(base, LANES)])


def kernel(x):
    mins, idxs = _phase1(x)
    return _phase2(mins, idxs)
